# baseline (speedup 1.0000x reference)
import functools

import jax
import jax.numpy as jnp
from jax import lax
from jax.experimental import pallas as pl
from jax.experimental.pallas import tpu as pltpu

_BETA = 0.25


def _tc_body(hfi_ref, wqTi_ref, bqi_ref, embi_ref, embTi_ref, wpTi_ref, bpi_ref,
             hfo_ref, wqTo_ref, bqo_ref, embo_ref, embTo_ref, wpTo_ref, bpo_ref,
             out_ref, loss_ref, gi_ref, go_ref, *, blk, K, Co_id):
    @pl.when(pl.program_id(0) == 0)
    def _make_g():
        gi = jnp.dot(embi_ref[...], wpTi_ref[...],
                     preferred_element_type=jnp.float32) + bpi_ref[...]
        go = jnp.dot(embo_ref[...], wpTo_ref[...],
                     preferred_element_type=jnp.float32) + bpo_ref[...]
        gi_ref[...] = gi.astype(jnp.bfloat16)
        go_ref[...] = go.astype(jnp.bfloat16)

    def branch(hf_ref, wqT_ref, bq_ref, emb_ref, embT_ref, g_ref, lo, hi):
        z = jnp.dot(hf_ref[...], wqT_ref[...],
                    preferred_element_type=jnp.float32) + bq_ref[...]
        ab = jnp.dot(z, embT_ref[...], preferred_element_type=jnp.float32)
        z2 = jnp.sum(z * z, axis=1, keepdims=True)
        e2 = jnp.sum(emb_ref[...] * emb_ref[...], axis=1)
        d = z2 + e2[None, :] - 2.0 * ab
        minval = jnp.min(d, axis=1)
        iota = lax.broadcasted_iota(jnp.int32, (blk, K), 1)
        idx = jnp.min(jnp.where(d <= minval[:, None], iota, K), axis=1)
        oh = (iota == idx[:, None]).astype(jnp.bfloat16)
        out_ref[:, lo:hi] = jnp.dot(
            oh, g_ref[...],
            preferred_element_type=jnp.float32).astype(jnp.bfloat16)
        return jnp.sum(minval)

    p_id = branch(hfi_ref, wqTi_ref, bqi_ref, embi_ref, embTi_ref, gi_ref,
                  0, Co_id)
    p_oth = branch(hfo_ref, wqTo_ref, bqo_ref, embo_ref, embTo_ref, go_ref,
                   Co_id, out_ref.shape[1])
    partial = jnp.stack([p_id, p_oth]).reshape(1, 2)

    @pl.when(pl.program_id(0) == 0)
    def _init():
        loss_ref[...] = partial

    @pl.when(pl.program_id(0) != 0)
    def _acc():
        loss_ref[...] += partial


def kernel(h_identity, h_others, W_quant_id, b_quant_id, codebook_id,
           W_post_id, b_post_id, W_quant_oth, b_quant_oth, codebook_oth,
           W_post_oth, b_post_oth, blk=2048):
    B, C_id, H, W = h_identity.shape
    C_oth = h_others.shape[1]
    N = B * H * W
    D_id = W_quant_id.shape[0]
    D_oth = W_quant_oth.shape[0]
    K = codebook_id.shape[0]
    Co_id = W_post_id.shape[0]
    Co_oth = W_post_oth.shape[0]
    hf_id = h_identity.transpose(0, 2, 3, 1).reshape(N, C_id)
    hf_oth = h_others.transpose(0, 2, 3, 1).reshape(N, C_oth)
    grid = N // blk
    full = lambda i: (0, 0)

    out_tok, loss_sums = pl.pallas_call(
        functools.partial(_tc_body, blk=blk, K=K, Co_id=Co_id),
        grid=(grid,),
        in_specs=[
            pl.BlockSpec((blk, C_id), lambda i: (i, 0)),
            pl.BlockSpec((C_id, D_id), full),
            pl.BlockSpec((1, D_id), full),
            pl.BlockSpec((K, D_id), full),
            pl.BlockSpec((D_id, K), full),
            pl.BlockSpec((D_id, Co_id), full),
            pl.BlockSpec((1, Co_id), full),
            pl.BlockSpec((blk, C_oth), lambda i: (i, 0)),
            pl.BlockSpec((C_oth, D_oth), full),
            pl.BlockSpec((1, D_oth), full),
            pl.BlockSpec((K, D_oth), full),
            pl.BlockSpec((D_oth, K), full),
            pl.BlockSpec((D_oth, Co_oth), full),
            pl.BlockSpec((1, Co_oth), full),
        ],
        out_specs=[
            pl.BlockSpec((blk, Co_id + Co_oth), lambda i: (i, 0)),
            pl.BlockSpec((1, 2), full),
        ],
        out_shape=[
            jax.ShapeDtypeStruct((N, Co_id + Co_oth), jnp.bfloat16),
            jax.ShapeDtypeStruct((1, 2), jnp.float32),
        ],
        scratch_shapes=[
            pltpu.VMEM((K, Co_id), jnp.bfloat16),
            pltpu.VMEM((K, Co_oth), jnp.bfloat16),
        ],
    )(hf_id, W_quant_id.T, b_quant_id[None, :], codebook_id, codebook_id.T,
      W_post_id.T, b_post_id[None, :],
      hf_oth, W_quant_oth.T, b_quant_oth[None, :], codebook_oth,
      codebook_oth.T, W_post_oth.T, b_post_oth[None, :])

    loss = (1.0 + _BETA) * (loss_sums[0, 0] / (N * D_id)
                            + loss_sums[0, 1] / (N * D_oth))
    Co = Co_id + Co_oth
    out = out_tok.reshape(B, H, W, Co).transpose(0, 3, 1, 2).astype(jnp.float32)
    return out, loss


# speedup vs baseline: 1.1118x; 1.1118x over previous
import functools

import jax
import jax.numpy as jnp
from jax import lax
from jax.experimental import pallas as pl
from jax.experimental.pallas import tpu as pltpu

_BETA = 0.25


def _tc_body(hfi_ref, wqTi_ref, bqi_ref, embi_ref, embTi_ref, wpTi_ref, bpi_ref,
             hfo_ref, wqTo_ref, bqo_ref, embo_ref, embTo_ref, wpTo_ref, bpo_ref,
             out_ref, loss_ref, gi_ref, go_ref, *, blk, K, Co_id):
    @pl.when(pl.program_id(0) == 0)
    def _make_g():
        gi = jnp.dot(embi_ref[...], wpTi_ref[...],
                     preferred_element_type=jnp.float32) + bpi_ref[...]
        go = jnp.dot(embo_ref[...], wpTo_ref[...],
                     preferred_element_type=jnp.float32) + bpo_ref[...]
        gi_ref[...] = gi.astype(jnp.bfloat16)
        go_ref[...] = go.astype(jnp.bfloat16)

    def branch(hf_ref, wqT_ref, bq_ref, emb_ref, embT_ref, g_ref, lo, hi):
        z = jnp.dot(hf_ref[...], wqT_ref[...],
                    preferred_element_type=jnp.float32) + bq_ref[...]
        ab = jnp.dot(z, embT_ref[...], preferred_element_type=jnp.float32)
        z2 = jnp.sum(z * z, axis=1, keepdims=True)
        e2 = jnp.sum(emb_ref[...] * emb_ref[...], axis=1)
        d = z2 + e2[None, :] - 2.0 * ab
        minval = jnp.min(d, axis=1)
        iota = lax.broadcasted_iota(jnp.int32, (blk, K), 1)
        idx = jnp.min(jnp.where(d <= minval[:, None], iota, K), axis=1)
        oh = (iota == idx[:, None]).astype(jnp.bfloat16)
        out_ref[:, lo:hi] = jnp.dot(oh, g_ref[...],
                                    preferred_element_type=jnp.float32)
        return jnp.sum(minval)

    p_id = branch(hfi_ref, wqTi_ref, bqi_ref, embi_ref, embTi_ref, gi_ref,
                  0, Co_id)
    p_oth = branch(hfo_ref, wqTo_ref, bqo_ref, embo_ref, embTo_ref, go_ref,
                   Co_id, out_ref.shape[1])
    partial = jnp.stack([p_id, p_oth]).reshape(1, 2)

    @pl.when(pl.program_id(0) == 0)
    def _init():
        loss_ref[...] = partial

    @pl.when(pl.program_id(0) != 0)
    def _acc():
        loss_ref[...] += partial


def kernel(h_identity, h_others, W_quant_id, b_quant_id, codebook_id,
           W_post_id, b_post_id, W_quant_oth, b_quant_oth, codebook_oth,
           W_post_oth, b_post_oth, blk=2048):
    B, C_id, H, W = h_identity.shape
    C_oth = h_others.shape[1]
    N = B * H * W
    D_id = W_quant_id.shape[0]
    D_oth = W_quant_oth.shape[0]
    K = codebook_id.shape[0]
    Co_id = W_post_id.shape[0]
    Co_oth = W_post_oth.shape[0]
    hf_id = h_identity.transpose(0, 2, 3, 1).reshape(N, C_id)
    hf_oth = h_others.transpose(0, 2, 3, 1).reshape(N, C_oth)
    grid = N // blk
    full = lambda i: (0, 0)

    out_tok, loss_sums = pl.pallas_call(
        functools.partial(_tc_body, blk=blk, K=K, Co_id=Co_id),
        grid=(grid,),
        in_specs=[
            pl.BlockSpec((blk, C_id), lambda i: (i, 0)),
            pl.BlockSpec((C_id, D_id), full),
            pl.BlockSpec((1, D_id), full),
            pl.BlockSpec((K, D_id), full),
            pl.BlockSpec((D_id, K), full),
            pl.BlockSpec((D_id, Co_id), full),
            pl.BlockSpec((1, Co_id), full),
            pl.BlockSpec((blk, C_oth), lambda i: (i, 0)),
            pl.BlockSpec((C_oth, D_oth), full),
            pl.BlockSpec((1, D_oth), full),
            pl.BlockSpec((K, D_oth), full),
            pl.BlockSpec((D_oth, K), full),
            pl.BlockSpec((D_oth, Co_oth), full),
            pl.BlockSpec((1, Co_oth), full),
        ],
        out_specs=[
            pl.BlockSpec((blk, Co_id + Co_oth), lambda i: (i, 0)),
            pl.BlockSpec((1, 2), full),
        ],
        out_shape=[
            jax.ShapeDtypeStruct((N, Co_id + Co_oth), jnp.float32),
            jax.ShapeDtypeStruct((1, 2), jnp.float32),
        ],
        scratch_shapes=[
            pltpu.VMEM((K, Co_id), jnp.bfloat16),
            pltpu.VMEM((K, Co_oth), jnp.bfloat16),
        ],
    )(hf_id, W_quant_id.T, b_quant_id[None, :], codebook_id, codebook_id.T,
      W_post_id.T, b_post_id[None, :],
      hf_oth, W_quant_oth.T, b_quant_oth[None, :], codebook_oth,
      codebook_oth.T, W_post_oth.T, b_post_oth[None, :])

    loss = (1.0 + _BETA) * (loss_sums[0, 0] / (N * D_id)
                            + loss_sums[0, 1] / (N * D_oth))
    Co = Co_id + Co_oth
    out = out_tok.reshape(B, H, W, Co).transpose(0, 3, 1, 2)
    return out, loss
